# baseline (speedup 1.0000x reference)
import jax
import jax.numpy as jnp
from jax import lax
from jax.experimental import pallas as pl
from jax.experimental.pallas import tpu as pltpu

HIDDEN = 4096
NUM_EXPERTS = 64
NUM_TOKENS = 32768

BT = 1024

_DN = (((1,), (1,)), ((), ()))


def _router_block(x_ref, w_ref, b_ref, o_ref):
    i = pl.program_id(0)
    o_ref[pl.ds(i * BT, BT), :] = (
        lax.dot_general(x_ref[...], w_ref[...], _DN,
                        preferred_element_type=jnp.float32)
        + b_ref[...]
    )


def kernel(x, W, b):
    b2 = b.reshape(1, NUM_EXPERTS)
    grid = (NUM_TOKENS // BT,)
    return pl.pallas_call(
        _router_block,
        grid=grid,
        in_specs=[
            pl.BlockSpec((BT, HIDDEN), lambda i: (i, 0)),
            pl.BlockSpec((NUM_EXPERTS, HIDDEN), lambda i: (0, 0)),
            pl.BlockSpec((1, NUM_EXPERTS), lambda i: (0, 0)),
        ],
        out_specs=pl.BlockSpec((NUM_TOKENS, NUM_EXPERTS), lambda i: (0, 0)),
        out_shape=jax.ShapeDtypeStruct((NUM_TOKENS, NUM_EXPERTS), jnp.float32),
        compiler_params=pltpu.CompilerParams(
            dimension_semantics=("arbitrary",),
            vmem_limit_bytes=63 * 1024 * 1024,
        ),
    )(x, W, b2)
